# baseline (speedup 1.0000x reference)
import functools

import jax
import jax.numpy as jnp
from jax import lax
from jax.experimental import pallas as pl
from jax.experimental.pallas import tpu as pltpu
from jax.experimental.pallas import tpu_sc as plsc

N = 10000
E = 320000
EMB = 32
H = 128
NG = 32
NF = 9
VOCAB = 100

NC, NS = 2, 16
NW = NC * NS
NPAD = 10240
NPT = NPAD // NW
NCS = 40
NCH = NPT // NCS
RPS = NPAD // NS
EC = 128
KCH = 80
EPT = EC * KCH
EPAD = NW * EPT
GS = 1

F32 = jnp.float32


def _mesh():
    return plsc.VectorSubcoreMesh(
        core_axis_name="c", subcore_axis_name="s", num_cores=NC, num_subcores=NS
    )


def _fill(ref, rows, width, value):
    vec = jnp.full((16,), value, F32)

    def body(r, _):
        for j in range(width // 16):
            ref[r, pl.ds(j * 16, 16)] = vec
        return 0

    lax.fori_loop(0, rows, body, 0)


NPASS = 4
NH = NPAD // NPASS
NRED = NH // 16


def _encdeg_body(xf_hbm, emb_hbm, dst_hbm, h0_hbm, deg_hbm,
                 xfb, dstb, accb, gbuf, hb, red, rb, acc, deg_s, sem):
    cid = lax.axis_index("c")
    sid = lax.axis_index("s")
    wid = sid * NC + cid

    pltpu.sync_copy(xf_hbm.at[wid], xfb)
    pltpu.sync_copy(dst_hbm.at[wid], dstb)

    nbase = wid * NPT

    def issue(c):
        return [
            pltpu.async_copy(
                emb_hbm.at[xfb.at[pl.ds(i * NPT + c * NCS, NCS)]], gbuf.at[i],
                sem,
            )
            for i in range(NF)
        ]

    cps = issue(0)

    zv = jnp.zeros((16,), F32)
    lanes = lax.iota(jnp.int32, 16) * NH
    ones = jnp.full((16,), 1.0, F32)

    for p in range(NPASS):
        def zero(i, _):
            hb[pl.ds(i * 16, 16)] = zv
            return 0

        lax.fori_loop(0, (16 * NH) // 16, zero, 0)

        base = p * NH

        def scan(g, _):
            d = dstb[pl.ds(g * 16, 16)] - base
            m = (d >= 0) & (d < NH)
            plsc.addupdate_scatter(hb, [d + lanes], ones, mask=m)
            return 0

        lax.fori_loop(0, EPT // 16, scan, 0)

        def reduce16(i, _):
            s = hb[pl.ds(i * 16, 16)]
            for c in range(1, 16):
                s = s + hb[pl.ds(c * NH + i * 16, 16)]
            red[pl.ds(i * 16, 16)] = s
            return 0

        lax.fori_loop(0, NRED, reduce16, 0)
        pltpu.sync_copy(red, deg_s.at[pl.ds(sid * NPAD + base, NH)])

    plsc.subcore_barrier()

    row0 = sid * RPS

    def zacc(i, _):
        acc[pl.ds(i * 16, 16)] = zv
        return 0

    lax.fori_loop(0, RPS // 16, zacc, 0)
    for t in range(NS):
        pltpu.sync_copy(deg_s.at[pl.ds(t * NPAD + row0, RPS)], rb)

        def accum(i, _):
            acc[pl.ds(i * 16, 16)] = (acc[pl.ds(i * 16, 16)]
                                      + rb[pl.ds(i * 16, 16)])
            return 0

        lax.fori_loop(0, RPS // 16, accum, 0)

    pltpu.sync_copy(acc, deg_hbm.at[pl.ds(cid * NPAD + row0, RPS)])

    for c in range(NCH):
        for cp in cps:
            cp.wait()

        def facc(r, _):
            for j in range(EMB // 16):
                s = gbuf[0, r, pl.ds(j * 16, 16)]
                for i in range(1, NF):
                    s = s + gbuf[i, r, pl.ds(j * 16, 16)]
                accb[r, pl.ds(j * 16, 16)] = s
            for j in range(EMB // 16, H // 16):
                accb[r, pl.ds(j * 16, 16)] = zv
            return 0

        lax.fori_loop(0, NCS, facc, 0)
        pltpu.sync_copy(accb, h0_hbm.at[pl.ds(nbase + c * NCS, NCS)])
        if c + 1 < NCH:
            cps = issue(c + 1)


_encdeg = functools.partial(
    pl.kernel,
    out_type=(
        jax.ShapeDtypeStruct((NPAD, H), F32),
        jax.ShapeDtypeStruct((NC * NPAD,), F32),
    ),
    mesh=_mesh(),
    compiler_params=pltpu.CompilerParams(needs_layout_passes=False),
    scratch_types=[
        pltpu.VMEM((NF * NPT,), jnp.int32),
        pltpu.VMEM((EPT,), jnp.int32),
        pltpu.VMEM((NCS, H), F32),
        pltpu.VMEM((NF, NCS, 128), F32),
        pltpu.VMEM((16 * NH,), F32),
        pltpu.VMEM((NH,), F32),
        pltpu.VMEM((RPS,), F32),
        pltpu.VMEM((RPS,), F32),
        pltpu.VMEM_SHARED((NS * NPAD,), F32),
        pltpu.SemaphoreType.DMA,
    ],
)(_encdeg_body)


IBLK = 16
NBLK = KCH // IBLK


def _scatter_body(w, g_hbm, srcf_hbm, dst_hbm, out_hbm,
                  srcb, dstb, rowa, rowb, s_s, sema, semb, semi, semsa, semsb):
    cid = lax.axis_index("c")
    sid = lax.axis_index("s")
    wid = sid * NC + cid

    _fill(rowa, EC, w, 0.0)
    row0 = sid * RPS
    for k in range(RPS // EC):
        pltpu.async_copy(rowa, s_s.at[pl.ds(row0 + k * EC, EC)], semb)

    pltpu.sync_copy(dst_hbm.at[wid], dstb)
    pltpu.sync_copy(srcf_hbm.at[wid * NBLK], srcb.at[pl.ds(0, IBLK * EC)])
    pltpu.async_copy(srcf_hbm.at[wid * NBLK + 1],
                     srcb.at[pl.ds(IBLK * EC, IBLK * EC)], semi)

    for k in range(RPS // EC):
        pltpu.make_async_copy(rowa, s_s.at[pl.ds(row0 + k * EC, EC)],
                              semb).wait()

    plsc.subcore_barrier()

    def src_ix(c):
        slot = lax.rem(lax.div(c, IBLK), 2)
        return srcb.at[pl.ds(slot * (IBLK * EC) + lax.rem(c, IBLK) * EC, EC)]

    pltpu.async_copy(g_hbm.at[srcb.at[pl.ds(0, EC)]], rowa, sema)
    pltpu.async_copy(g_hbm.at[srcb.at[pl.ds(EC, EC)]], rowb, semb)

    def step(m, _):
        c0, c1, c2, c3 = 2 * m, 2 * m + 1, 2 * m + 2, 2 * m + 3
        pltpu.make_async_copy(g_hbm.at[src_ix(c0)], rowa, sema).wait()
        pltpu.async_copy(rowa, s_s.at[dstb.at[c0]], semsa, add=True)
        pltpu.make_async_copy(g_hbm.at[src_ix(c1)], rowb, semb).wait()
        pltpu.async_copy(rowb, s_s.at[dstb.at[c1]], semsb, add=True)

        @pl.when(m < KCH // 2 - 1)
        def _():
            @pl.when(lax.rem(c2, IBLK) == 0)
            def _():
                blk = lax.div(c2, IBLK)
                slot = lax.rem(blk, 2)
                pltpu.make_async_copy(
                    srcf_hbm.at[wid * NBLK + blk],
                    srcb.at[pl.ds(slot * (IBLK * EC), IBLK * EC)], semi,
                ).wait()

                @pl.when(blk + 1 < NBLK)
                def _():
                    nslot = lax.rem(blk + 1, 2)
                    pltpu.async_copy(
                        srcf_hbm.at[wid * NBLK + blk + 1],
                        srcb.at[pl.ds(nslot * (IBLK * EC), IBLK * EC)], semi)

            pltpu.make_async_copy(rowa, s_s.at[dstb.at[c0]], semsa).wait()
            pltpu.async_copy(g_hbm.at[src_ix(c2)], rowa, sema)
            pltpu.make_async_copy(rowb, s_s.at[dstb.at[c1]], semsb).wait()
            pltpu.async_copy(g_hbm.at[src_ix(c3)], rowb, semb)

        return 0

    lax.fori_loop(0, KCH // 2, step, 0)

    pltpu.make_async_copy(rowa, s_s.at[dstb.at[KCH - 2]], semsa).wait()
    pltpu.make_async_copy(rowb, s_s.at[dstb.at[KCH - 1]], semsb).wait()

    plsc.subcore_barrier()
    pltpu.sync_copy(s_s.at[pl.ds(row0, RPS)],
                    out_hbm.at[pl.ds(cid * NPAD + row0, RPS)])


def _make_scatter(w):
    return functools.partial(
        pl.kernel,
        out_type=jax.ShapeDtypeStruct((NC * NPAD, w), F32),
        mesh=_mesh(),
        scratch_types=[
            pltpu.VMEM((2 * IBLK * EC,), jnp.int32),
            pltpu.VMEM((KCH, EC), jnp.int32),
            pltpu.VMEM((EC, w), F32),
            pltpu.VMEM((EC, w), F32),
            pltpu.VMEM_SHARED((NPAD, w), F32),
            pltpu.SemaphoreType.DMA,
            pltpu.SemaphoreType.DMA,
            pltpu.SemaphoreType.DMA,
            pltpu.SemaphoreType.DMA,
            pltpu.SemaphoreType.DMA,
        ],
    )(functools.partial(_scatter_body, w))


_scatter128 = _make_scatter(H)


BLK = 512
GRID = NPAD // BLK


def _norm_of(d0, d1):
    return lax.rsqrt(d0[:, 0:1] + d1[:, 0:1] + 1.0)


def _enc_norm_body(h0, d0, d1, g0):
    g0[...] = h0[...] * _norm_of(d0, d1)


def _layer_body(s0, s1, gp, d0, d1, w, b, gn):
    nrm = _norm_of(d0, d1)
    agg = (s0[...] + s1[...] + gp[...]) * nrm
    h = jnp.dot(agg, w[...], preferred_element_type=F32) + b[...]
    gn[...] = jnp.maximum(h, 0.0) * nrm


def _final_body(s0, s1, gp, d0, d1, w3, b3, oh, wout, bout, out, sums, cnts):
    i = pl.program_id(0)

    @pl.when(i == 0)
    def _():
        sums[...] = jnp.zeros((NG, H), F32)
        cnts[...] = jnp.zeros((NG, 128), F32)

    nrm = _norm_of(d0, d1)
    agg = (s0[...] + s1[...] + gp[...]) * nrm
    h3 = jnp.dot(agg, w3[...], preferred_element_type=F32) + b3[...]
    ohb = oh[...]
    sums[...] += lax.dot_general(ohb, h3, (((0,), (0,)), ((), ())),
                                 preferred_element_type=F32)
    cnts[...] += lax.dot_general(ohb, jnp.ones((BLK, 128), F32),
                                 (((0,), (0,)), ((), ())),
                                 preferred_element_type=F32)

    @pl.when(i == GRID - 1)
    def _():
        pooled = sums[...] / jnp.maximum(cnts[...], 1.0)
        out[...] = jnp.dot(pooled, wout[...], preferred_element_type=F32) + bout[...]


def _row_spec(w):
    return pl.BlockSpec((BLK, w), lambda i: (i, 0))


def _full_spec(r, c):
    return pl.BlockSpec((r, c), lambda i: (0, 0))


def _enc_norm(h0, d0, d1):
    return pl.pallas_call(
        _enc_norm_body,
        grid=(GRID,),
        in_specs=[_row_spec(H), _row_spec(16), _row_spec(16)],
        out_specs=_row_spec(H),
        out_shape=jax.ShapeDtypeStruct((NPAD, H), F32),
    )(h0, d0, d1)


def _layer(s0, s1, gp, d0, d1, w, b):
    win = gp.shape[1]
    return pl.pallas_call(
        _layer_body,
        grid=(GRID,),
        in_specs=[_row_spec(win), _row_spec(win), _row_spec(win),
                  _row_spec(16), _row_spec(16),
                  _full_spec(win, H), _full_spec(1, H)],
        out_specs=_row_spec(H),
        out_shape=jax.ShapeDtypeStruct((NPAD, H), F32),
    )(s0, s1, gp, d0, d1, w, b)


def _final(s0, s1, gp, d0, d1, w3, b3, oh, wout, bout):
    return pl.pallas_call(
        _final_body,
        grid=(GRID,),
        in_specs=[_row_spec(H), _row_spec(H), _row_spec(H),
                  _row_spec(16), _row_spec(16),
                  _full_spec(H, H), _full_spec(1, H),
                  _row_spec(NG),
                  _full_spec(H, 10), _full_spec(1, 10)],
        out_specs=_full_spec(NG, 10),
        out_shape=jax.ShapeDtypeStruct((NG, 10), F32),
        scratch_shapes=[pltpu.VMEM((NG, H), F32), pltpu.VMEM((NG, 128), F32)],
    )(s0, s1, gp, d0, d1, w3, b3, oh, wout, bout)


def kernel(x, edge_index, batch_ids, emb_tables, W1, b1, W2, b2, W3, b3,
           Wout, bout):
    xfill = (jnp.arange((NPAD - N) * NF, dtype=jnp.int32) % VOCAB
             ).reshape(NPAD - N, NF)
    x_pad = jnp.concatenate([x, xfill], axis=0)
    xf = x_pad.T + (jnp.arange(NF, dtype=jnp.int32) * VOCAB)[:, None]
    xf3 = xf.reshape(NF, NW, NPT).transpose(1, 0, 2).reshape(NW, NF * NPT)
    emb_flat = jnp.pad(emb_tables.reshape(NF * VOCAB, EMB),
                       ((0, 0), (0, 128 - EMB)))

    epad = N + (jnp.arange(EPAD - E, dtype=jnp.int32) % (NPAD - N))
    srcf = jnp.concatenate([edge_index[0], epad]).reshape(NW * NBLK, IBLK * EC)
    dst3 = jnp.concatenate([edge_index[1], epad]).reshape(NW, KCH, EC)
    dstf = dst3.reshape(NW, EPT)

    bid_pad = jnp.concatenate(
        [batch_ids, jnp.full((NPAD - N,), NG, jnp.int32)])
    oh = (bid_pad[:, None] == jnp.arange(NG, dtype=jnp.int32)[None, :]
          ).astype(F32)

    W1p = jnp.pad(W1, ((0, H - EMB), (0, 0)))
    b1r, b2r, b3r = b1.reshape(1, H), b2.reshape(1, H), b3.reshape(1, H)
    boutr = bout.reshape(1, 10)

    h0, degf = _encdeg(xf3, emb_flat, dstf)
    d0 = jnp.broadcast_to(degf[:NPAD, None], (NPAD, 16))
    d1 = jnp.broadcast_to(degf[NPAD:, None], (NPAD, 16))

    g0 = _enc_norm(h0, d0, d1)

    sp = _scatter128(g0, srcf, dst3)
    g1 = _layer(sp[:NPAD], sp[NPAD:], g0, d0, d1, W1p, b1r)

    sp = _scatter128(g1, srcf, dst3)
    g2 = _layer(sp[:NPAD], sp[NPAD:], g1, d0, d1, W2, b2r)

    sp = _scatter128(g2, srcf, dst3)
    out = _final(sp[:NPAD], sp[NPAD:], g2, d0, d1, W3, b3r, oh, Wout, boutr)
    return out


# speedup vs baseline: 1.0720x; 1.0720x over previous
import functools

import jax
import jax.numpy as jnp
from jax import lax
from jax.experimental import pallas as pl
from jax.experimental.pallas import tpu as pltpu
from jax.experimental.pallas import tpu_sc as plsc

N = 10000
E = 320000
EMB = 32
H = 128
NG = 32
NF = 9
VOCAB = 100

NC, NS = 2, 16
NW = NC * NS
NPAD = 10240
NPT = NPAD // NW
NCS = 16
NCH = NPT // NCS
RPS = NPAD // NS
EC = 128
KCH = 80
EPT = EC * KCH
EPAD = NW * EPT
GS = 1

F32 = jnp.float32


def _mesh():
    return plsc.VectorSubcoreMesh(
        core_axis_name="c", subcore_axis_name="s", num_cores=NC, num_subcores=NS
    )


def _fill(ref, rows, width, value):
    vec = jnp.full((16,), value, F32)

    def body(r, _):
        for j in range(width // 16):
            ref[r, pl.ds(j * 16, 16)] = vec
        return 0

    lax.fori_loop(0, rows, body, 0)


NPASS = 4
NH = NPAD // NPASS
NRED = NH // 16


def _encdeg_body(xf_hbm, emb_hbm, dst_hbm, h0_hbm, deg_hbm,
                 xfb, dstb, accb, gbuf, hb, red, rb, acc, deg_s, sem0, sem1):
    cid = lax.axis_index("c")
    sid = lax.axis_index("s")
    wid = sid * NC + cid

    pltpu.sync_copy(xf_hbm.at[wid], xfb)
    pltpu.sync_copy(dst_hbm.at[wid], dstb)

    nbase = wid * NPT
    sems = (sem0, sem1)

    def issue(c):
        slot = c % 2
        return [
            pltpu.async_copy(
                emb_hbm.at[xfb.at[pl.ds(i * NPT + c * NCS, NCS)]],
                gbuf.at[slot, i], sems[slot],
            )
            for i in range(NF)
        ]

    cps = issue(0)
    nxt = issue(1)

    zv = jnp.zeros((16,), F32)
    lanes = lax.iota(jnp.int32, 16) * NH
    ones = jnp.full((16,), 1.0, F32)

    for p in range(NPASS):
        def zero(i, _):
            hb[pl.ds(i * 16, 16)] = zv
            return 0

        lax.fori_loop(0, (16 * NH) // 16, zero, 0)

        base = p * NH

        def scan(g, _):
            d = dstb[pl.ds(g * 16, 16)] - base
            m = (d >= 0) & (d < NH)
            plsc.addupdate_scatter(hb, [d + lanes], ones, mask=m)
            return 0

        lax.fori_loop(0, EPT // 16, scan, 0)

        def reduce16(i, _):
            s = hb[pl.ds(i * 16, 16)]
            for c in range(1, 16):
                s = s + hb[pl.ds(c * NH + i * 16, 16)]
            red[pl.ds(i * 16, 16)] = s
            return 0

        lax.fori_loop(0, NRED, reduce16, 0)
        pltpu.sync_copy(red, deg_s.at[pl.ds(sid * NPAD + base, NH)])

    plsc.subcore_barrier()

    row0 = sid * RPS

    def zacc(i, _):
        acc[pl.ds(i * 16, 16)] = zv
        return 0

    lax.fori_loop(0, RPS // 16, zacc, 0)
    for t in range(NS):
        pltpu.sync_copy(deg_s.at[pl.ds(t * NPAD + row0, RPS)], rb)

        def accum(i, _):
            acc[pl.ds(i * 16, 16)] = (acc[pl.ds(i * 16, 16)]
                                      + rb[pl.ds(i * 16, 16)])
            return 0

        lax.fori_loop(0, RPS // 16, accum, 0)

    pltpu.sync_copy(acc, deg_hbm.at[pl.ds(cid * NPAD + row0, RPS)])

    for c in range(NCH):
        for cp in cps:
            cp.wait()
        slot = c % 2

        def facc(r, _):
            for j in range(EMB // 16):
                s = gbuf[slot, 0, r, pl.ds(j * 16, 16)]
                for i in range(1, NF):
                    s = s + gbuf[slot, i, r, pl.ds(j * 16, 16)]
                accb[r, pl.ds(j * 16, 16)] = s
            for j in range(EMB // 16, H // 16):
                accb[r, pl.ds(j * 16, 16)] = zv
            return 0

        lax.fori_loop(0, NCS, facc, 0)
        cps = nxt
        if c + 2 < NCH:
            nxt = issue(c + 2)
        pltpu.sync_copy(accb, h0_hbm.at[pl.ds(nbase + c * NCS, NCS)])


_encdeg = functools.partial(
    pl.kernel,
    out_type=(
        jax.ShapeDtypeStruct((NPAD, H), F32),
        jax.ShapeDtypeStruct((NC * NPAD,), F32),
    ),
    mesh=_mesh(),
    compiler_params=pltpu.CompilerParams(needs_layout_passes=False),
    scratch_types=[
        pltpu.VMEM((NF * NPT,), jnp.int32),
        pltpu.VMEM((EPT,), jnp.int32),
        pltpu.VMEM((NCS, H), F32),
        pltpu.VMEM((2, NF, NCS, 128), F32),
        pltpu.VMEM((16 * NH,), F32),
        pltpu.VMEM((NH,), F32),
        pltpu.VMEM((RPS,), F32),
        pltpu.VMEM((RPS,), F32),
        pltpu.VMEM_SHARED((NS * NPAD,), F32),
        pltpu.SemaphoreType.DMA,
        pltpu.SemaphoreType.DMA,
    ],
)(_encdeg_body)


IBLK = 16
NBLK = KCH // IBLK


def _scatter_body(w, g_hbm, srcf_hbm, dst_hbm, out_hbm,
                  srcb, dstb, rowa, rowb, s_s, sema, semb, semi):
    cid = lax.axis_index("c")
    sid = lax.axis_index("s")
    wid = sid * NC + cid

    _fill(rowa, EC, w, 0.0)
    row0 = sid * RPS
    for k in range(RPS // EC):
        pltpu.async_copy(rowa, s_s.at[pl.ds(row0 + k * EC, EC)], semb)

    pltpu.sync_copy(dst_hbm.at[wid], dstb)
    pltpu.sync_copy(srcf_hbm.at[wid * NBLK], srcb.at[pl.ds(0, IBLK * EC)])
    pltpu.async_copy(srcf_hbm.at[wid * NBLK + 1],
                     srcb.at[pl.ds(IBLK * EC, IBLK * EC)], semi)

    for k in range(RPS // EC):
        pltpu.make_async_copy(rowa, s_s.at[pl.ds(row0 + k * EC, EC)],
                              semb).wait()

    plsc.subcore_barrier()

    def src_ix(c):
        slot = lax.rem(lax.div(c, IBLK), 2)
        return srcb.at[pl.ds(slot * (IBLK * EC) + lax.rem(c, IBLK) * EC, EC)]

    pltpu.async_copy(g_hbm.at[srcb.at[pl.ds(0, EC)]], rowa, sema)

    def step(m, _):
        c0, c1, c2 = 2 * m, 2 * m + 1, 2 * m + 2
        pltpu.make_async_copy(g_hbm.at[src_ix(c0)], rowa, sema).wait()
        pltpu.async_copy(g_hbm.at[src_ix(c1)], rowb, semb)
        pltpu.sync_copy(rowa, s_s.at[dstb.at[c0]], add=True)
        pltpu.make_async_copy(g_hbm.at[src_ix(c1)], rowb, semb).wait()

        @pl.when(m < KCH // 2 - 1)
        def _():
            @pl.when(lax.rem(c2, IBLK) == 0)
            def _():
                blk = lax.div(c2, IBLK)
                slot = lax.rem(blk, 2)
                pltpu.make_async_copy(
                    srcf_hbm.at[wid * NBLK + blk],
                    srcb.at[pl.ds(slot * (IBLK * EC), IBLK * EC)], semi,
                ).wait()

                @pl.when(blk + 1 < NBLK)
                def _():
                    nslot = lax.rem(blk + 1, 2)
                    pltpu.async_copy(
                        srcf_hbm.at[wid * NBLK + blk + 1],
                        srcb.at[pl.ds(nslot * (IBLK * EC), IBLK * EC)], semi)

            pltpu.async_copy(g_hbm.at[src_ix(c2)], rowa, sema)

        pltpu.sync_copy(rowb, s_s.at[dstb.at[c1]], add=True)
        return 0

    lax.fori_loop(0, KCH // 2, step, 0)

    plsc.subcore_barrier()
    pltpu.sync_copy(s_s.at[pl.ds(row0, RPS)],
                    out_hbm.at[pl.ds(cid * NPAD + row0, RPS)])


def _make_scatter(w):
    return functools.partial(
        pl.kernel,
        out_type=jax.ShapeDtypeStruct((NC * NPAD, w), F32),
        mesh=_mesh(),
        scratch_types=[
            pltpu.VMEM((2 * IBLK * EC,), jnp.int32),
            pltpu.VMEM((KCH, EC), jnp.int32),
            pltpu.VMEM((EC, w), F32),
            pltpu.VMEM((EC, w), F32),
            pltpu.VMEM_SHARED((NPAD, w), F32),
            pltpu.SemaphoreType.DMA,
            pltpu.SemaphoreType.DMA,
            pltpu.SemaphoreType.DMA,
        ],
    )(functools.partial(_scatter_body, w))


_scatter128 = _make_scatter(H)


BLK = 512
GRID = NPAD // BLK


def _norm_of(d0, d1):
    return lax.rsqrt(d0[:, 0:1] + d1[:, 0:1] + 1.0)


def _enc_norm_body(h0, d0, d1, g0):
    g0[...] = h0[...] * _norm_of(d0, d1)


def _layer_body(s0, s1, gp, d0, d1, w, b, gn):
    nrm = _norm_of(d0, d1)
    agg = (s0[...] + s1[...] + gp[...]) * nrm
    h = jnp.dot(agg, w[...], preferred_element_type=F32) + b[...]
    gn[...] = jnp.maximum(h, 0.0) * nrm


def _final_body(s0, s1, gp, d0, d1, w3, b3, oh, wout, bout, out, sums, cnts):
    i = pl.program_id(0)

    @pl.when(i == 0)
    def _():
        sums[...] = jnp.zeros((NG, H), F32)
        cnts[...] = jnp.zeros((NG, 128), F32)

    nrm = _norm_of(d0, d1)
    agg = (s0[...] + s1[...] + gp[...]) * nrm
    h3 = jnp.dot(agg, w3[...], preferred_element_type=F32) + b3[...]
    ohb = oh[...]
    sums[...] += lax.dot_general(ohb, h3, (((0,), (0,)), ((), ())),
                                 preferred_element_type=F32)
    cnts[...] += lax.dot_general(ohb, jnp.ones((BLK, 128), F32),
                                 (((0,), (0,)), ((), ())),
                                 preferred_element_type=F32)

    @pl.when(i == GRID - 1)
    def _():
        pooled = sums[...] / jnp.maximum(cnts[...], 1.0)
        out[...] = jnp.dot(pooled, wout[...], preferred_element_type=F32) + bout[...]


def _row_spec(w):
    return pl.BlockSpec((BLK, w), lambda i: (i, 0))


def _full_spec(r, c):
    return pl.BlockSpec((r, c), lambda i: (0, 0))


def _enc_norm(h0, d0, d1):
    return pl.pallas_call(
        _enc_norm_body,
        grid=(GRID,),
        in_specs=[_row_spec(H), _row_spec(16), _row_spec(16)],
        out_specs=_row_spec(H),
        out_shape=jax.ShapeDtypeStruct((NPAD, H), F32),
    )(h0, d0, d1)


def _layer(s0, s1, gp, d0, d1, w, b):
    win = gp.shape[1]
    return pl.pallas_call(
        _layer_body,
        grid=(GRID,),
        in_specs=[_row_spec(win), _row_spec(win), _row_spec(win),
                  _row_spec(16), _row_spec(16),
                  _full_spec(win, H), _full_spec(1, H)],
        out_specs=_row_spec(H),
        out_shape=jax.ShapeDtypeStruct((NPAD, H), F32),
    )(s0, s1, gp, d0, d1, w, b)


def _final(s0, s1, gp, d0, d1, w3, b3, oh, wout, bout):
    return pl.pallas_call(
        _final_body,
        grid=(GRID,),
        in_specs=[_row_spec(H), _row_spec(H), _row_spec(H),
                  _row_spec(16), _row_spec(16),
                  _full_spec(H, H), _full_spec(1, H),
                  _row_spec(NG),
                  _full_spec(H, 10), _full_spec(1, 10)],
        out_specs=_full_spec(NG, 10),
        out_shape=jax.ShapeDtypeStruct((NG, 10), F32),
        scratch_shapes=[pltpu.VMEM((NG, H), F32), pltpu.VMEM((NG, 128), F32)],
    )(s0, s1, gp, d0, d1, w3, b3, oh, wout, bout)


def kernel(x, edge_index, batch_ids, emb_tables, W1, b1, W2, b2, W3, b3,
           Wout, bout):
    xfill = (jnp.arange((NPAD - N) * NF, dtype=jnp.int32) % VOCAB
             ).reshape(NPAD - N, NF)
    x_pad = jnp.concatenate([x, xfill], axis=0)
    xf = x_pad.T + (jnp.arange(NF, dtype=jnp.int32) * VOCAB)[:, None]
    xf3 = xf.reshape(NF, NW, NPT).transpose(1, 0, 2).reshape(NW, NF * NPT)
    emb_flat = jnp.pad(emb_tables.reshape(NF * VOCAB, EMB),
                       ((0, 0), (0, 128 - EMB)))

    epad = N + (jnp.arange(EPAD - E, dtype=jnp.int32) % (NPAD - N))
    srcf = jnp.concatenate([edge_index[0], epad]).reshape(NW * NBLK, IBLK * EC)
    dst3 = jnp.concatenate([edge_index[1], epad]).reshape(NW, KCH, EC)
    dstf = dst3.reshape(NW, EPT)

    bid_pad = jnp.concatenate(
        [batch_ids, jnp.full((NPAD - N,), NG, jnp.int32)])
    oh = (bid_pad[:, None] == jnp.arange(NG, dtype=jnp.int32)[None, :]
          ).astype(F32)

    W1p = jnp.pad(W1, ((0, H - EMB), (0, 0)))
    b1r, b2r, b3r = b1.reshape(1, H), b2.reshape(1, H), b3.reshape(1, H)
    boutr = bout.reshape(1, 10)

    h0, degf = _encdeg(xf3, emb_flat, dstf)
    d0 = jnp.broadcast_to(degf[:NPAD, None], (NPAD, 16))
    d1 = jnp.broadcast_to(degf[NPAD:, None], (NPAD, 16))

    g0 = _enc_norm(h0, d0, d1)

    sp = _scatter128(g0, srcf, dst3)
    g1 = _layer(sp[:NPAD], sp[NPAD:], g0, d0, d1, W1p, b1r)

    sp = _scatter128(g1, srcf, dst3)
    g2 = _layer(sp[:NPAD], sp[NPAD:], g1, d0, d1, W2, b2r)

    sp = _scatter128(g2, srcf, dst3)
    out = _final(sp[:NPAD], sp[NPAD:], g2, d0, d1, W3, b3r, oh, Wout, boutr)
    return out


# speedup vs baseline: 1.1927x; 1.1126x over previous
import functools

import jax
import jax.numpy as jnp
from jax import lax
from jax.experimental import pallas as pl
from jax.experimental.pallas import tpu as pltpu
from jax.experimental.pallas import tpu_sc as plsc

N = 10000
E = 320000
EMB = 32
H = 128
NG = 32
NF = 9
VOCAB = 100

NC, NS = 2, 16
NW = NC * NS
NPAD = 10240
NPT = NPAD // NW
NCS = 16
NCH = NPT // NCS
RPS = NPAD // NS
EC = 128
KCH = 80
EPT = EC * KCH
EPAD = NW * EPT
GS = 1

F32 = jnp.float32


def _mesh():
    return plsc.VectorSubcoreMesh(
        core_axis_name="c", subcore_axis_name="s", num_cores=NC, num_subcores=NS
    )


def _fill(ref, rows, width, value):
    vec = jnp.full((16,), value, F32)

    def body(r, _):
        for j in range(width // 16):
            ref[r, pl.ds(j * 16, 16)] = vec
        return 0

    lax.fori_loop(0, rows, body, 0)


NPASS = 4
NH = NPAD // NPASS
NRED = NH // 16


def _encdeg_body(xf_hbm, emb_hbm, dst_hbm, h0_hbm, deg_hbm,
                 xfb, dstb, accb, gbuf, hb, red, rb, acc, deg_s, sem0, sem1):
    cid = lax.axis_index("c")
    sid = lax.axis_index("s")
    wid = sid * NC + cid

    pltpu.sync_copy(xf_hbm.at[wid], xfb)
    pltpu.sync_copy(dst_hbm.at[wid], dstb)

    nbase = wid * NPT
    sems = (sem0, sem1)

    def issue(c):
        slot = c % 2
        return [
            pltpu.async_copy(
                emb_hbm.at[xfb.at[pl.ds(i * NPT + c * NCS, NCS)]],
                gbuf.at[slot, i], sems[slot],
            )
            for i in range(NF)
        ]

    cps = issue(0)
    nxt = issue(1)

    zv = jnp.zeros((16,), F32)
    lanes = lax.iota(jnp.int32, 16) * NH
    ones = jnp.full((16,), 1.0, F32)

    for p in range(NPASS):
        def zero(i, _):
            hb[pl.ds(i * 16, 16)] = zv
            return 0

        lax.fori_loop(0, (16 * NH) // 16, zero, 0)

        base = p * NH

        def scan(g, _):
            d = dstb[pl.ds(g * 16, 16)] - base
            m = (d >= 0) & (d < NH)
            plsc.addupdate_scatter(hb, [d + lanes], ones, mask=m)
            return 0

        lax.fori_loop(0, EPT // 16, scan, 0)

        def reduce16(i, _):
            s = hb[pl.ds(i * 16, 16)]
            for c in range(1, 16):
                s = s + hb[pl.ds(c * NH + i * 16, 16)]
            red[pl.ds(i * 16, 16)] = s
            return 0

        lax.fori_loop(0, NRED, reduce16, 0)
        pltpu.sync_copy(red, deg_s.at[pl.ds(sid * NPAD + base, NH)])

    plsc.subcore_barrier()

    row0 = sid * RPS

    def zacc(i, _):
        acc[pl.ds(i * 16, 16)] = zv
        return 0

    lax.fori_loop(0, RPS // 16, zacc, 0)
    for t in range(NS):
        pltpu.sync_copy(deg_s.at[pl.ds(t * NPAD + row0, RPS)], rb)

        def accum(i, _):
            acc[pl.ds(i * 16, 16)] = (acc[pl.ds(i * 16, 16)]
                                      + rb[pl.ds(i * 16, 16)])
            return 0

        lax.fori_loop(0, RPS // 16, accum, 0)

    pltpu.sync_copy(acc, deg_hbm.at[pl.ds(cid * NPAD + row0, RPS)])

    for c in range(NCH):
        for cp in cps:
            cp.wait()
        slot = c % 2

        def facc(r, _):
            for j in range(EMB // 16):
                s = gbuf[slot, 0, r, pl.ds(j * 16, 16)]
                for i in range(1, NF):
                    s = s + gbuf[slot, i, r, pl.ds(j * 16, 16)]
                accb[r, pl.ds(j * 16, 16)] = s
            for j in range(EMB // 16, H // 16):
                accb[r, pl.ds(j * 16, 16)] = zv
            return 0

        lax.fori_loop(0, NCS, facc, 0)
        cps = nxt
        if c + 2 < NCH:
            nxt = issue(c + 2)
        pltpu.sync_copy(accb, h0_hbm.at[pl.ds(nbase + c * NCS, NCS)])


_encdeg = functools.partial(
    pl.kernel,
    out_type=(
        jax.ShapeDtypeStruct((NPAD, H), F32),
        jax.ShapeDtypeStruct((NC * NPAD,), F32),
    ),
    mesh=_mesh(),
    compiler_params=pltpu.CompilerParams(needs_layout_passes=False),
    scratch_types=[
        pltpu.VMEM((NF * NPT,), jnp.int32),
        pltpu.VMEM((EPT,), jnp.int32),
        pltpu.VMEM((NCS, H), F32),
        pltpu.VMEM((2, NF, NCS, 128), F32),
        pltpu.VMEM((16 * NH,), F32),
        pltpu.VMEM((NH,), F32),
        pltpu.VMEM((RPS,), F32),
        pltpu.VMEM((RPS,), F32),
        pltpu.VMEM_SHARED((NS * NPAD,), F32),
        pltpu.SemaphoreType.DMA,
        pltpu.SemaphoreType.DMA,
    ],
)(_encdeg_body)


IBLK = 16
NBLK = KCH // IBLK


def _scatter_body(w, g_hbm, srcf_hbm, dst_hbm, out_hbm,
                  srcb, dstb, rowa, rowb, s_s, sema, semb, semi):
    cid = lax.axis_index("c")
    sid = lax.axis_index("s")
    wid = sid * NC + cid

    _fill(rowa, EC, w, 0.0)
    row0 = sid * RPS
    for k in range(RPS // EC):
        pltpu.async_copy(rowa, s_s.at[pl.ds(row0 + k * EC, EC)], semb)

    pltpu.sync_copy(dst_hbm.at[wid], dstb)
    pltpu.sync_copy(srcf_hbm.at[wid * NBLK], srcb.at[pl.ds(0, IBLK * EC)])
    pltpu.async_copy(srcf_hbm.at[wid * NBLK + 1],
                     srcb.at[pl.ds(IBLK * EC, IBLK * EC)], semi)

    for k in range(RPS // EC):
        pltpu.make_async_copy(rowa, s_s.at[pl.ds(row0 + k * EC, EC)],
                              semb).wait()

    plsc.subcore_barrier()

    def src_ix(c):
        slot = lax.rem(lax.div(c, IBLK), 2)
        return srcb.at[pl.ds(slot * (IBLK * EC) + lax.rem(c, IBLK) * EC, EC)]

    pltpu.async_copy(g_hbm.at[srcb.at[pl.ds(0, EC)]], rowa, sema)

    def step(m, _):
        c0, c1, c2 = 2 * m, 2 * m + 1, 2 * m + 2
        pltpu.async_copy(g_hbm.at[src_ix(c1)], rowb, semb)
        pltpu.make_async_copy(g_hbm.at[src_ix(c0)], rowa, sema).wait()
        pltpu.sync_copy(rowa, s_s.at[dstb.at[c0]], add=True)

        @pl.when(m < KCH // 2 - 1)
        def _():
            @pl.when(lax.rem(c2, IBLK) == 0)
            def _():
                blk = lax.div(c2, IBLK)
                slot = lax.rem(blk, 2)
                pltpu.make_async_copy(
                    srcf_hbm.at[wid * NBLK + blk],
                    srcb.at[pl.ds(slot * (IBLK * EC), IBLK * EC)], semi,
                ).wait()

                @pl.when(blk + 1 < NBLK)
                def _():
                    nslot = lax.rem(blk + 1, 2)
                    pltpu.async_copy(
                        srcf_hbm.at[wid * NBLK + blk + 1],
                        srcb.at[pl.ds(nslot * (IBLK * EC), IBLK * EC)], semi)

            pltpu.async_copy(g_hbm.at[src_ix(c2)], rowa, sema)

        pltpu.make_async_copy(g_hbm.at[src_ix(c1)], rowb, semb).wait()
        pltpu.sync_copy(rowb, s_s.at[dstb.at[c1]], add=True)
        return 0

    lax.fori_loop(0, KCH // 2, step, 0)

    plsc.subcore_barrier()
    pltpu.sync_copy(s_s.at[pl.ds(row0, RPS)],
                    out_hbm.at[pl.ds(cid * NPAD + row0, RPS)])


def _make_scatter(w):
    return functools.partial(
        pl.kernel,
        out_type=jax.ShapeDtypeStruct((NC * NPAD, w), F32),
        mesh=_mesh(),
        scratch_types=[
            pltpu.VMEM((2 * IBLK * EC,), jnp.int32),
            pltpu.VMEM((KCH, EC), jnp.int32),
            pltpu.VMEM((EC, w), F32),
            pltpu.VMEM((EC, w), F32),
            pltpu.VMEM_SHARED((NPAD, w), F32),
            pltpu.SemaphoreType.DMA,
            pltpu.SemaphoreType.DMA,
            pltpu.SemaphoreType.DMA,
        ],
    )(functools.partial(_scatter_body, w))


_scatter128 = _make_scatter(H)


BLK = 512
GRID = NPAD // BLK


def _norm_of(d0, d1):
    return lax.rsqrt(d0[:, 0:1] + d1[:, 0:1] + 1.0)


def _enc_norm_body(h0, d0, d1, g0):
    g0[...] = h0[...] * _norm_of(d0, d1)


def _layer_body(s0, s1, gp, d0, d1, w, b, gn):
    nrm = _norm_of(d0, d1)
    agg = (s0[...] + s1[...] + gp[...]) * nrm
    h = jnp.dot(agg, w[...], preferred_element_type=F32) + b[...]
    gn[...] = jnp.maximum(h, 0.0) * nrm


def _final_body(s0, s1, gp, d0, d1, w3, b3, oh, wout, bout, out, sums, cnts):
    i = pl.program_id(0)

    @pl.when(i == 0)
    def _():
        sums[...] = jnp.zeros((NG, H), F32)
        cnts[...] = jnp.zeros((NG, 128), F32)

    nrm = _norm_of(d0, d1)
    agg = (s0[...] + s1[...] + gp[...]) * nrm
    h3 = jnp.dot(agg, w3[...], preferred_element_type=F32) + b3[...]
    ohb = oh[...]
    sums[...] += lax.dot_general(ohb, h3, (((0,), (0,)), ((), ())),
                                 preferred_element_type=F32)
    cnts[...] += lax.dot_general(ohb, jnp.ones((BLK, 128), F32),
                                 (((0,), (0,)), ((), ())),
                                 preferred_element_type=F32)

    @pl.when(i == GRID - 1)
    def _():
        pooled = sums[...] / jnp.maximum(cnts[...], 1.0)
        out[...] = jnp.dot(pooled, wout[...], preferred_element_type=F32) + bout[...]


def _row_spec(w):
    return pl.BlockSpec((BLK, w), lambda i: (i, 0))


def _full_spec(r, c):
    return pl.BlockSpec((r, c), lambda i: (0, 0))


def _enc_norm(h0, d0, d1):
    return pl.pallas_call(
        _enc_norm_body,
        grid=(GRID,),
        in_specs=[_row_spec(H), _row_spec(16), _row_spec(16)],
        out_specs=_row_spec(H),
        out_shape=jax.ShapeDtypeStruct((NPAD, H), F32),
    )(h0, d0, d1)


def _layer(s0, s1, gp, d0, d1, w, b):
    win = gp.shape[1]
    return pl.pallas_call(
        _layer_body,
        grid=(GRID,),
        in_specs=[_row_spec(win), _row_spec(win), _row_spec(win),
                  _row_spec(16), _row_spec(16),
                  _full_spec(win, H), _full_spec(1, H)],
        out_specs=_row_spec(H),
        out_shape=jax.ShapeDtypeStruct((NPAD, H), F32),
    )(s0, s1, gp, d0, d1, w, b)


def _final(s0, s1, gp, d0, d1, w3, b3, oh, wout, bout):
    return pl.pallas_call(
        _final_body,
        grid=(GRID,),
        in_specs=[_row_spec(H), _row_spec(H), _row_spec(H),
                  _row_spec(16), _row_spec(16),
                  _full_spec(H, H), _full_spec(1, H),
                  _row_spec(NG),
                  _full_spec(H, 10), _full_spec(1, 10)],
        out_specs=_full_spec(NG, 10),
        out_shape=jax.ShapeDtypeStruct((NG, 10), F32),
        scratch_shapes=[pltpu.VMEM((NG, H), F32), pltpu.VMEM((NG, 128), F32)],
    )(s0, s1, gp, d0, d1, w3, b3, oh, wout, bout)


def kernel(x, edge_index, batch_ids, emb_tables, W1, b1, W2, b2, W3, b3,
           Wout, bout):
    xfill = (jnp.arange((NPAD - N) * NF, dtype=jnp.int32) % VOCAB
             ).reshape(NPAD - N, NF)
    x_pad = jnp.concatenate([x, xfill], axis=0)
    xf = x_pad.T + (jnp.arange(NF, dtype=jnp.int32) * VOCAB)[:, None]
    xf3 = xf.reshape(NF, NW, NPT).transpose(1, 0, 2).reshape(NW, NF * NPT)
    emb_flat = jnp.pad(emb_tables.reshape(NF * VOCAB, EMB),
                       ((0, 0), (0, 128 - EMB)))

    epad = N + (jnp.arange(EPAD - E, dtype=jnp.int32) % (NPAD - N))
    srcf = jnp.concatenate([edge_index[0], epad]).reshape(NW * NBLK, IBLK * EC)
    dst3 = jnp.concatenate([edge_index[1], epad]).reshape(NW, KCH, EC)
    dstf = dst3.reshape(NW, EPT)

    bid_pad = jnp.concatenate(
        [batch_ids, jnp.full((NPAD - N,), NG, jnp.int32)])
    oh = (bid_pad[:, None] == jnp.arange(NG, dtype=jnp.int32)[None, :]
          ).astype(F32)

    W1p = jnp.pad(W1, ((0, H - EMB), (0, 0)))
    b1r, b2r, b3r = b1.reshape(1, H), b2.reshape(1, H), b3.reshape(1, H)
    boutr = bout.reshape(1, 10)

    h0, degf = _encdeg(xf3, emb_flat, dstf)
    d0 = jnp.broadcast_to(degf[:NPAD, None], (NPAD, 16))
    d1 = jnp.broadcast_to(degf[NPAD:, None], (NPAD, 16))

    g0 = _enc_norm(h0, d0, d1)

    sp = _scatter128(g0, srcf, dst3)
    g1 = _layer(sp[:NPAD], sp[NPAD:], g0, d0, d1, W1p, b1r)

    sp = _scatter128(g1, srcf, dst3)
    g2 = _layer(sp[:NPAD], sp[NPAD:], g1, d0, d1, W2, b2r)

    sp = _scatter128(g2, srcf, dst3)
    out = _final(sp[:NPAD], sp[NPAD:], g2, d0, d1, W3, b3r, oh, Wout, boutr)
    return out
